# baseline (speedup 1.0000x reference)
import jax
import jax.numpy as jnp
from jax.experimental import pallas as pl

_BS = 512


def _body(x_ref, p_ref, o_ref):
    x = x_ref[...]
    p = p_ref[...]
    o_ref[...] = x + jnp.where(x != 0.0, p, 0.0)


def kernel(inputs, pos_table):
    B, L, D = inputs.shape
    return pl.pallas_call(
        _body,
        grid=(L // _BS,),
        in_specs=[
            pl.BlockSpec((4, _BS, D), lambda s: (0, s, 0)),
            pl.BlockSpec((_BS, D), lambda s: (s, 0)),
        ],
        out_specs=pl.BlockSpec((4, _BS, D), lambda s: (0, s, 0)),
        out_shape=jax.ShapeDtypeStruct((B, L, D), inputs.dtype),
    )(inputs, pos_table)


# speedup vs baseline: 1.0210x; 1.0210x over previous
import jax
import jax.numpy as jnp
from jax.experimental import pallas as pl

_BS = 1024


def _body(x_ref, p_ref, o_ref):
    x = x_ref[...]
    p = p_ref[...]
    o_ref[...] = x + jnp.where(x != 0.0, p, 0.0)


def kernel(inputs, pos_table):
    B, L, D = inputs.shape
    return pl.pallas_call(
        _body,
        grid=(L // _BS, B // 2),
        in_specs=[
            pl.BlockSpec((2, _BS, D), lambda s, b: (b, s, 0)),
            pl.BlockSpec((_BS, D), lambda s, b: (s, 0)),
        ],
        out_specs=pl.BlockSpec((2, _BS, D), lambda s, b: (b, s, 0)),
        out_shape=jax.ShapeDtypeStruct((B, L, D), inputs.dtype),
    )(inputs, pos_table)
